# baseline (speedup 1.0000x reference)
import jax
import jax.numpy as jnp
from jax.experimental import pallas as pl
from jax.experimental.pallas import tpu as pltpu
from jax.experimental.pallas import tpu_sc as plsc

_NWORK = 32
_NBUF = 4
_BSTEP = 4


def kernel(token_ids, matrix):
    b, s = token_ids.shape
    n, d = matrix.shape
    steps = b // (_NWORK * _BSTEP)
    batches_per_worker = b // _NWORK
    indices = token_ids.astype(jnp.int32).reshape(_NWORK, steps, _BSTEP, s)

    mesh = plsc.VectorSubcoreMesh(
        core_axis_name="core", subcore_axis_name="subcore"
    )

    @pl.kernel(
        out_type=jax.ShapeDtypeStruct((b, s, d), matrix.dtype),
        mesh=mesh,
        scratch_types=[
            pltpu.VMEM((steps, _BSTEP, s), jnp.int32),
            pltpu.VMEM((_NBUF, _BSTEP, s, d), matrix.dtype),
            pltpu.SemaphoreType.DMA((_NBUF,)),
            pltpu.SemaphoreType.DMA((_NBUF,)),
            pltpu.SemaphoreType.DMA,
        ],
    )
    def gather_kernel(x_hbm, i_hbm, o_hbm, idx_ref, buf_ref, gsem, wsem, isem):
        core = jax.lax.axis_index("core")
        sub = jax.lax.axis_index("subcore")
        w = core * 16 + sub
        base = w * batches_per_worker

        pltpu.async_copy(i_hbm.at[w], idx_ref, isem).wait()

        def start_gather(bu, step):
            return [
                pltpu.async_copy(
                    x_hbm.at[idx_ref.at[step, j]],
                    buf_ref.at[bu, j],
                    gsem.at[bu],
                )
                for j in range(_BSTEP)
            ]

        def start_writeback(bu, step):
            return pltpu.async_copy(
                buf_ref.at[bu],
                o_hbm.at[pl.ds(base + step * _BSTEP, _BSTEP)],
                wsem.at[bu],
            )

        gathers = [start_gather(k, k) for k in range(_NBUF)]
        writebacks = [None] * _NBUF
        for step in range(steps):
            bu = step % _NBUF
            for h in gathers[bu]:
                h.wait()
            writebacks[bu] = start_writeback(bu, step)
            nxt = step + _NBUF
            if nxt < steps:
                writebacks[bu].wait()
                gathers[bu] = start_gather(bu, nxt)
        for k in range(max(0, steps - _NBUF), steps):
            writebacks[k % _NBUF].wait()

    return gather_kernel(matrix, indices)
